# baseline (speedup 1.0000x reference)
import functools

import jax
import jax.numpy as jnp
from jax import lax
from jax.experimental import pallas as pl
from jax.experimental.pallas import tpu as pltpu
from jax.experimental.pallas import tpu_sc as plsc

B = 4096
F = 32
NC = 2
NS = 16
NW = NC * NS
BPW = B // NW
L = 16
GROUPS = BPW // L


def _sc_bias_sum(ub_g, ib_g):
  mesh = plsc.VectorSubcoreMesh(
      core_axis_name="c", subcore_axis_name="s",
      num_cores=NC, num_subcores=NS)

  @functools.partial(
      pl.kernel,
      out_type=jax.ShapeDtypeStruct((B,), jnp.float32),
      mesh=mesh,
      compiler_params=pltpu.CompilerParams(
          needs_layout_passes=False, use_tc_tiling_on_sc=False),
      scratch_types=[
          pltpu.VMEM((BPW,), jnp.float32),
          pltpu.VMEM((BPW,), jnp.float32),
          pltpu.VMEM((BPW,), jnp.float32),
      ],
  )
  def sc_kernel(ub_hbm, ib_hbm, s_out, ub_v, ib_v, s_v):
    wid = lax.axis_index("s") * NC + lax.axis_index("c")
    base = wid * BPW

    pltpu.sync_copy(ub_hbm.at[pl.ds(base, BPW)], ub_v)
    pltpu.sync_copy(ib_hbm.at[pl.ds(base, BPW)], ib_v)

    def group_body(g, carry):
      sl = pl.ds(g * L, L)
      s_v[sl] = ub_v[sl] + ib_v[sl]
      return carry

    lax.fori_loop(0, GROUPS, group_body, 0)
    pltpu.sync_copy(s_v, s_out.at[pl.ds(base, BPW)])

  return sc_kernel(ub_g, ib_g)


def _tc_dot_broadcast_body(ufT_ref, itT_ref, s_ref, out_ref):
  prod = ufT_ref[...] * itT_ref[...]
  dotrow = jnp.sum(prod, axis=0, keepdims=True)
  out_ref[...] = s_ref[...] + dotrow


def kernel(user_ids, item_ids, user_factors, item_factors, user_bias,
           item_bias):
  ufg = jnp.take(user_factors, user_ids, axis=0, mode="clip")
  itg = jnp.take(item_factors, item_ids, axis=0, mode="clip")
  zeros = jnp.zeros_like(user_ids)
  ub_g = user_bias[user_ids, zeros]
  ib_g = item_bias[item_ids, zeros]
  s = _sc_bias_sum(ub_g, ib_g)
  s2d = s.reshape(B, 1)

  BM = 256
  out = pl.pallas_call(
      _tc_dot_broadcast_body,
      grid=(B // BM,),
      in_specs=[
          pl.BlockSpec((F, B), lambda i: (0, 0)),
          pl.BlockSpec((F, B), lambda i: (0, 0)),
          pl.BlockSpec((BM, 1), lambda i: (i, 0)),
      ],
      out_specs=pl.BlockSpec((BM, B), lambda i: (i, 0)),
      out_shape=jax.ShapeDtypeStruct((B, B), jnp.float32),
      compiler_params=pltpu.CompilerParams(
          dimension_semantics=("parallel",)),
  )(ufg.T, itg.T, s2d)
  return out


# speedup vs baseline: 1.0149x; 1.0149x over previous
import functools

import jax
import jax.numpy as jnp
from jax import lax
from jax.experimental import pallas as pl
from jax.experimental.pallas import tpu as pltpu
from jax.experimental.pallas import tpu_sc as plsc

B = 4096
F = 32
NC = 2
NS = 16
NW = NC * NS
BPW = B // NW
L = 16
GROUPS = BPW // L


def _sc_bias_sum(ub_g, ib_g):
  mesh = plsc.VectorSubcoreMesh(
      core_axis_name="c", subcore_axis_name="s",
      num_cores=NC, num_subcores=NS)

  @functools.partial(
      pl.kernel,
      out_type=jax.ShapeDtypeStruct((B,), jnp.float32),
      mesh=mesh,
      compiler_params=pltpu.CompilerParams(
          needs_layout_passes=False, use_tc_tiling_on_sc=False),
      scratch_types=[
          pltpu.VMEM((BPW,), jnp.float32),
          pltpu.VMEM((BPW,), jnp.float32),
          pltpu.VMEM((BPW,), jnp.float32),
      ],
  )
  def sc_kernel(ub_hbm, ib_hbm, s_out, ub_v, ib_v, s_v):
    wid = lax.axis_index("s") * NC + lax.axis_index("c")
    base = wid * BPW

    pltpu.sync_copy(ub_hbm.at[pl.ds(base, BPW)], ub_v)
    pltpu.sync_copy(ib_hbm.at[pl.ds(base, BPW)], ib_v)

    def group_body(g, carry):
      sl = pl.ds(g * L, L)
      s_v[sl] = ub_v[sl] + ib_v[sl]
      return carry

    lax.fori_loop(0, GROUPS, group_body, 0)
    pltpu.sync_copy(s_v, s_out.at[pl.ds(base, BPW)])

  return sc_kernel(ub_g, ib_g)


def _tc_dot_broadcast_body(ufT_ref, itT_ref, s_ref, out_ref):
  prod = ufT_ref[...] * itT_ref[...]
  dotrow = jnp.sum(prod, axis=0, keepdims=True)
  out_ref[...] = s_ref[...] + dotrow


def kernel(user_ids, item_ids, user_factors, item_factors, user_bias,
           item_bias):
  ufg = jnp.take(user_factors, user_ids, axis=0, mode="clip")
  itg = jnp.take(item_factors, item_ids, axis=0, mode="clip")
  zeros = jnp.zeros_like(user_ids)
  ub_g = user_bias[user_ids, zeros]
  ib_g = item_bias[item_ids, zeros]
  s = _sc_bias_sum(ub_g, ib_g)
  s2d = s.reshape(B, 1)

  BM = 512
  out = pl.pallas_call(
      _tc_dot_broadcast_body,
      grid=(B // BM,),
      in_specs=[
          pl.BlockSpec((F, B), lambda i: (0, 0)),
          pl.BlockSpec((F, B), lambda i: (0, 0)),
          pl.BlockSpec((BM, 1), lambda i: (i, 0)),
      ],
      out_specs=pl.BlockSpec((BM, B), lambda i: (i, 0)),
      out_shape=jax.ShapeDtypeStruct((B, B), jnp.float32),
      compiler_params=pltpu.CompilerParams(
          dimension_semantics=("parallel",)),
  )(ufg.T, itg.T, s2d)
  return out
